# baseline (speedup 1.0000x reference)
import functools
import jax
import jax.numpy as jnp
from jax import lax
from jax.experimental import pallas as pl
from jax.experimental.pallas import tpu as pltpu

TOPK = 128
NCAND = 256
INV_LOG10 = 0.43429448190325176


def _loss_body(vr_ref, ir_ref, sr_ref, vc_ref, ic_ref, sc_ref, out_ref):
    vr = vr_ref[...]
    ir = ir_ref[...]
    sr = sr_ref[...]
    vc = vc_ref[...]
    ic = ic_ref[...]
    sc = sc_ref[...]

    beats_cr = jnp.logical_or(vr > vc, jnp.logical_and(vr == vc, ir < ic))
    r_col = jnp.sum(beats_cr.astype(jnp.float32), axis=1, keepdims=True)
    beats_rc = jnp.logical_or(vc > vr, jnp.logical_and(vc == vr, ic < ir))
    r_row = jnp.sum(beats_rc.astype(jnp.float32), axis=0, keepdims=True)

    kr = lax.broadcasted_iota(jnp.int32, (NCAND, TOPK), 1).astype(jnp.float32)
    P = (r_col == kr).astype(jnp.float32)
    kc = lax.broadcasted_iota(jnp.int32, (TOPK, NCAND), 0).astype(jnp.float32)
    Q = (kc == r_row).astype(jnp.float32)

    sv_row = jnp.dot(vr, P, preferred_element_type=jnp.float32)
    ss_row = jnp.dot(sr, P, preferred_element_type=jnp.float32)
    sv_col = jnp.dot(Q, vc, preferred_element_type=jnp.float32)
    ss_col = jnp.dot(Q, sc, preferred_element_type=jnp.float32)

    i_r = lax.broadcasted_iota(jnp.int32, (TOPK, TOPK), 1)
    i_c = lax.broadcasted_iota(jnp.int32, (TOPK, TOPK), 0)
    mask = (i_r > i_c).astype(jnp.float32)

    pnl_d = (sv_col - sv_row) * mask
    npd = jnp.log(pnl_d + 1.0) * INV_LOG10
    x = (ss_col - ss_row) * mask
    bce = jnp.maximum(x, 0.0) - x * mask + jnp.log(1.0 + jnp.exp(-jnp.abs(x)))
    total = jnp.sum(npd * bce) * (1.0 / TOPK)
    out_ref[...] = jnp.broadcast_to(total, (1, 1))


@jax.jit
def _loss_stage(vals, idxs, scrs):
    vr = vals.reshape(1, NCAND)
    ir = idxs.reshape(1, NCAND).astype(jnp.int32)
    sr = scrs.reshape(1, NCAND)
    vc = vals.reshape(NCAND, 1)
    ic = idxs.reshape(NCAND, 1).astype(jnp.int32)
    sc = scrs.reshape(NCAND, 1)
    out = pl.pallas_call(
        _loss_body,
        out_shape=jax.ShapeDtypeStruct((1, 1), jnp.float32),
    )(vr, ir, sr, vc, ic, sc)
    return out.reshape(())


def kernel(scores, target, pnl):
    del target
    vals, idxs = jax.lax.top_k(pnl, TOPK)
    scr = scores[idxs]
    pad_v = jnp.full((NCAND - TOPK,), -jnp.inf, jnp.float32)
    pad_i = (1 << 20) + jnp.arange(NCAND - TOPK, dtype=jnp.int32)
    pad_s = jnp.zeros((NCAND - TOPK,), jnp.float32)
    vals = jnp.concatenate([vals, pad_v])
    idxs = jnp.concatenate([idxs.astype(jnp.int32), pad_i])
    scr = jnp.concatenate([scr, pad_s])
    return _loss_stage(vals, idxs, scr)


# speedup vs baseline: 16.8623x; 16.8623x over previous
import functools
import jax
import jax.numpy as jnp
from jax import lax
from jax.experimental import pallas as pl
from jax.experimental.pallas import tpu as pltpu
from jax.experimental.pallas import tpu_sc as plsc

TOPK = 128
NCAND = 256
INV_LOG10 = 0.43429448190325176

N = 1_000_000
PADN = 1_048_576
CHUNK = PADN // 32
NV = CHUNK // 16
SEG_ELEMS = 256
NSEG = CHUNK // SEG_ELEMS
GRP_SEGS = 16
NGRP = NSEG // GRP_SEGS
NEG = jnp.float32(-jnp.inf)
BIGPOS = jnp.int32(1 << 30)


def _lane():
    return lax.iota(jnp.int32, 16)


def _take(v, idx):
    return v.at[idx].get(mode="promise_in_bounds")


def _extract_at(vec, l, fill):
    lane = _lane()
    m = jnp.where(lane == l, vec, fill)
    for k in (8, 4, 2, 1):
        m = jnp.maximum(m, _take(m, (lane + k) & 15))
    return m[0]


def _argmax_tree(v, p):
    lane = _lane()
    cv, cp = v, p
    for k in (8, 4, 2, 1):
        sh = (lane + k) & 15
        pv = _take(cv, sh)
        pp = _take(cp, sh)
        better = jnp.logical_or(pv > cv, jnp.logical_and(pv == cv, pp < cp))
        cv = jnp.where(better, pv, cv)
        cp = jnp.where(better, pp, cp)
    vs = cv[0]
    ps = cp[0]
    match = jnp.logical_and(v == vs, p == ps)
    lid = jnp.where(match, lane, 16)
    for k in (8, 4, 2, 1):
        lid = jnp.minimum(lid, _take(lid, (lane + k) & 15))
    return vs, ps, lid[0]


def _argmax_tree3(v, p, i):
    lane = _lane()
    cv, cp, ci = v, p, i
    for k in (8, 4, 2, 1):
        sh = (lane + k) & 15
        pv = _take(cv, sh)
        pp = _take(cp, sh)
        pi = _take(ci, sh)
        better = jnp.logical_or(pv > cv, jnp.logical_and(pv == cv, pp < cp))
        cv = jnp.where(better, pv, cv)
        cp = jnp.where(better, pp, cp)
        ci = jnp.where(better, pi, ci)
    return cv[0], cp[0], ci[0]


def _scan_pairs(val_ref, pos_ref, base, count):
    def body(j, carry):
        mv, mp = carry
        v = val_ref[pl.ds((base + j) * 16, 16)]
        p = pos_ref[pl.ds((base + j) * 16, 16)]
        upd = jnp.logical_or(v > mv, jnp.logical_and(v == mv, p < mp))
        return jnp.where(upd, v, mv), jnp.where(upd, p, mp)
    return lax.fori_loop(0, count, body, (jnp.full((16,), NEG),
                                          jnp.full((16,), BIGPOS)))


def _sc_body(pnl_hbm, scores_hbm, ov_hbm, oi_hbm, os_hbm,
             data, mv1, mp1, gv2, gp2, tvv, tvi,
             cmv, cmi, cmsi, outv, outi, cidx, scrv, svv, svi, sem):
    c = lax.axis_index("c")
    s = lax.axis_index("s")
    w = c * 16 + s
    tilebase = w * CHUNK
    lane = _lane()

    pltpu.sync_copy(pnl_hbm.at[pl.ds(w * CHUNK, CHUNK)], data)

    def seg_build(seg, _):
        def inner(j, carry):
            mv, mp = carry
            pos0 = seg * SEG_ELEMS + j * 16
            v = data[pl.ds(pos0, 16)]
            p = pos0 + lane
            upd = jnp.logical_or(v > mv, jnp.logical_and(v == mv, p < mp))
            return jnp.where(upd, v, mv), jnp.where(upd, p, mp)
        mv, mp = lax.fori_loop(0, SEG_ELEMS // 16, inner,
                               (jnp.full((16,), NEG), jnp.full((16,), BIGPOS)))
        mv1[pl.ds(seg * 16, 16)] = mv
        mp1[pl.ds(seg * 16, 16)] = mp
        return 0
    lax.fori_loop(0, NSEG, seg_build, 0)

    def grp_build(g, _):
        gv, gp = _scan_pairs(mv1, mp1, g * GRP_SEGS, GRP_SEGS)
        gv2[pl.ds(g * 16, 16)] = gv
        gp2[pl.ds(g * 16, 16)] = gp
        return 0
    lax.fori_loop(0, NGRP, grp_build, 0)

    def extract(t, carry):
        wv, wi = carry
        m3v, m3p = _scan_pairs(gv2, gp2, 0, NGRP)
        vs, ps, _ = _argmax_tree(m3v, m3p)
        vo = (ps >> 4) << 4
        lv = ps & 15
        dv = data[pl.ds(vo, 16)]
        data[pl.ds(vo, 16)] = jnp.where(lane == lv, NEG, dv)
        sstar = ps >> 8
        gstar = ps >> 12
        def seg_rescan(j, carry):
            mv, mp = carry
            pos0 = sstar * SEG_ELEMS + j * 16
            v = data[pl.ds(pos0, 16)]
            p = pos0 + lane
            upd = jnp.logical_or(v > mv, jnp.logical_and(v == mv, p < mp))
            return jnp.where(upd, v, mv), jnp.where(upd, p, mp)
        mv, mp = lax.fori_loop(0, SEG_ELEMS // 16, seg_rescan,
                               (jnp.full((16,), NEG), jnp.full((16,), BIGPOS)))
        mv1[pl.ds(sstar * 16, 16)] = mv
        mp1[pl.ds(sstar * 16, 16)] = mp
        gv, gp = _scan_pairs(mv1, mp1, gstar * GRP_SEGS, GRP_SEGS)
        gv2[pl.ds(gstar * 16, 16)] = gv
        gp2[pl.ds(gstar * 16, 16)] = gp
        slot = t & 15
        wv = jnp.where(slot == 0, jnp.full((16,), NEG), wv)
        wi = jnp.where(slot == 0, jnp.zeros((16,), jnp.int32), wi)
        wv = jnp.where(lane == slot, vs, wv)
        wi = jnp.where(lane == slot, tilebase + ps, wi)
        tvv[pl.ds((t >> 4) << 4, 16)] = wv
        tvi[pl.ds((t >> 4) << 4, 16)] = wi
        return wv, wi
    lax.fori_loop(0, TOPK, extract,
                  (jnp.full((16,), NEG), jnp.zeros((16,), jnp.int32)))

    pltpu.sync_copy(tvv, svv.at[pl.ds(s * TOPK, TOPK)])
    pltpu.sync_copy(tvi, svi.at[pl.ds(s * TOPK, TOPK)])
    plsc.subcore_barrier()

    pltpu.sync_copy(svv, cmv)
    pltpu.sync_copy(svi, cmi)

    def mseg_build(seg, _):
        def inner(j, carry):
            mv, mp, mi = carry
            pos0 = seg * 256 + j * 16
            v = cmv[pl.ds(pos0, 16)]
            i = cmi[pl.ds(pos0, 16)]
            p = pos0 + lane
            upd = jnp.logical_or(v > mv, jnp.logical_and(v == mv, p < mp))
            return (jnp.where(upd, v, mv), jnp.where(upd, p, mp),
                    jnp.where(upd, i, mi))
        mv, mp, mi = lax.fori_loop(0, 16, inner,
                                   (jnp.full((16,), NEG),
                                    jnp.full((16,), BIGPOS),
                                    jnp.zeros((16,), jnp.int32)))
        gv2[pl.ds(seg * 16, 16)] = mv
        gp2[pl.ds(seg * 16, 16)] = mp
        cmsi[pl.ds(seg * 16, 16)] = mi
        return 0
    lax.fori_loop(0, 8, mseg_build, 0)

    def merge(t, carry):
        wv, wi = carry
        def topscan(j, carry):
            mv, mp, mi = carry
            v = gv2[pl.ds(j * 16, 16)]
            p = gp2[pl.ds(j * 16, 16)]
            i = cmsi[pl.ds(j * 16, 16)]
            upd = jnp.logical_or(v > mv, jnp.logical_and(v == mv, p < mp))
            return (jnp.where(upd, v, mv), jnp.where(upd, p, mp),
                    jnp.where(upd, i, mi))
        m3v, m3p, m3i = lax.fori_loop(0, 8, topscan,
                                      (jnp.full((16,), NEG),
                                       jnp.full((16,), BIGPOS),
                                       jnp.zeros((16,), jnp.int32)))
        vs, ps, is_ = _argmax_tree3(m3v, m3p, m3i)
        vo = (ps >> 4) << 4
        lv = ps & 15
        dv = cmv[pl.ds(vo, 16)]
        cmv[pl.ds(vo, 16)] = jnp.where(lane == lv, NEG, dv)
        sstar = ps >> 8
        def mseg_rescan(j, carry):
            mv, mp, mi = carry
            pos0 = sstar * 256 + j * 16
            v = cmv[pl.ds(pos0, 16)]
            i = cmi[pl.ds(pos0, 16)]
            p = pos0 + lane
            upd = jnp.logical_or(v > mv, jnp.logical_and(v == mv, p < mp))
            return (jnp.where(upd, v, mv), jnp.where(upd, p, mp),
                    jnp.where(upd, i, mi))
        mv, mp, mi = lax.fori_loop(0, 16, mseg_rescan,
                                   (jnp.full((16,), NEG),
                                    jnp.full((16,), BIGPOS),
                                    jnp.zeros((16,), jnp.int32)))
        gv2[pl.ds(sstar * 16, 16)] = mv
        gp2[pl.ds(sstar * 16, 16)] = mp
        cmsi[pl.ds(sstar * 16, 16)] = mi
        slot = t & 15
        wv = jnp.where(slot == 0, jnp.full((16,), NEG), wv)
        wi = jnp.where(slot == 0, jnp.zeros((16,), jnp.int32), wi)
        wv = jnp.where(lane == slot, vs, wv)
        wi = jnp.where(lane == slot, is_, wi)
        outv[pl.ds((t >> 4) << 4, 16)] = wv
        outi[pl.ds((t >> 4) << 4, 16)] = wi
        return wv, wi
    lax.fori_loop(0, TOPK, merge,
                  (jnp.full((16,), NEG), jnp.zeros((16,), jnp.int32)))

    def clampi(j, _):
        cidx[pl.ds(j * 16, 16)] = jnp.minimum(outi[pl.ds(j * 16, 16)], N - 1)
        return 0
    lax.fori_loop(0, TOPK // 16, clampi, 0)

    @pl.when(s == 0)
    def _():
        pltpu.async_copy(scores_hbm.at[cidx], scrv, sem).wait()
        pltpu.sync_copy(outv, ov_hbm.at[pl.ds(c * TOPK, TOPK)])
        pltpu.sync_copy(outi, oi_hbm.at[pl.ds(c * TOPK, TOPK)])
        pltpu.sync_copy(scrv, os_hbm.at[pl.ds(c * TOPK, TOPK)])


@jax.jit
def _candidate_stage(pnl_padded, scores):
    mesh = plsc.VectorSubcoreMesh(core_axis_name="c", subcore_axis_name="s")
    f = pl.kernel(
        _sc_body,
        out_type=(
            jax.ShapeDtypeStruct((NCAND,), jnp.float32),
            jax.ShapeDtypeStruct((NCAND,), jnp.int32),
            jax.ShapeDtypeStruct((NCAND,), jnp.float32),
        ),
        mesh=mesh,
        scratch_types=[
            pltpu.VMEM((CHUNK,), jnp.float32),
            pltpu.VMEM((NSEG * 16,), jnp.float32),
            pltpu.VMEM((NSEG * 16,), jnp.int32),
            pltpu.VMEM((NGRP * 16,), jnp.float32),
            pltpu.VMEM((NGRP * 16,), jnp.int32),
            pltpu.VMEM((TOPK,), jnp.float32),
            pltpu.VMEM((TOPK,), jnp.int32),
            pltpu.VMEM((16 * TOPK,), jnp.float32),
            pltpu.VMEM((16 * TOPK,), jnp.int32),
            pltpu.VMEM((NGRP * 16,), jnp.int32),
            pltpu.VMEM((TOPK,), jnp.float32),
            pltpu.VMEM((TOPK,), jnp.int32),
            pltpu.VMEM((TOPK,), jnp.int32),
            pltpu.VMEM((TOPK,), jnp.float32),
            pltpu.VMEM_SHARED((16 * TOPK,), jnp.float32),
            pltpu.VMEM_SHARED((16 * TOPK,), jnp.int32),
            pltpu.SemaphoreType.DMA,
        ],
    )
    return f(pnl_padded, scores)


def _loss_body(vr_ref, ir_ref, sr_ref, vc_ref, ic_ref, sc_ref, out_ref):
    vr = vr_ref[...]
    ir = ir_ref[...]
    sr = sr_ref[...]
    vc = vc_ref[...]
    ic = ic_ref[...]
    sc = sc_ref[...]

    beats_cr = jnp.logical_or(vr > vc, jnp.logical_and(vr == vc, ir < ic))
    r_col = jnp.sum(beats_cr.astype(jnp.float32), axis=1, keepdims=True)
    beats_rc = jnp.logical_or(vc > vr, jnp.logical_and(vc == vr, ic < ir))
    r_row = jnp.sum(beats_rc.astype(jnp.float32), axis=0, keepdims=True)

    kr = lax.broadcasted_iota(jnp.int32, (NCAND, TOPK), 1).astype(jnp.float32)
    P = (r_col == kr).astype(jnp.float32)
    kc = lax.broadcasted_iota(jnp.int32, (TOPK, NCAND), 0).astype(jnp.float32)
    Q = (kc == r_row).astype(jnp.float32)

    sv_row = jnp.dot(vr, P, preferred_element_type=jnp.float32)
    ss_row = jnp.dot(sr, P, preferred_element_type=jnp.float32)
    sv_col = jnp.dot(Q, vc, preferred_element_type=jnp.float32)
    ss_col = jnp.dot(Q, sc, preferred_element_type=jnp.float32)

    i_r = lax.broadcasted_iota(jnp.int32, (TOPK, TOPK), 1)
    i_c = lax.broadcasted_iota(jnp.int32, (TOPK, TOPK), 0)
    mask = (i_r > i_c).astype(jnp.float32)

    pnl_d = (sv_col - sv_row) * mask
    npd = jnp.log(pnl_d + 1.0) * INV_LOG10
    x = (ss_col - ss_row) * mask
    bce = jnp.maximum(x, 0.0) - x * mask + jnp.log(1.0 + jnp.exp(-jnp.abs(x)))
    total = jnp.sum(npd * bce) * (1.0 / TOPK)
    out_ref[...] = jnp.broadcast_to(total, (1, 1))


@jax.jit
def _loss_stage(vals, idxs, scrs):
    vr = vals.reshape(1, NCAND)
    ir = idxs.reshape(1, NCAND).astype(jnp.int32)
    sr = scrs.reshape(1, NCAND)
    vc = vals.reshape(NCAND, 1)
    ic = idxs.reshape(NCAND, 1).astype(jnp.int32)
    sc = scrs.reshape(NCAND, 1)
    out = pl.pallas_call(
        _loss_body,
        out_shape=jax.ShapeDtypeStruct((1, 1), jnp.float32),
    )(vr, ir, sr, vc, ic, sc)
    return out.reshape(())


def kernel(scores, target, pnl):
    del target
    pnl_padded = jnp.concatenate(
        [pnl, jnp.full((PADN - N,), -jnp.inf, jnp.float32)])
    vals, idxs, scrs = _candidate_stage(pnl_padded, scores)
    return _loss_stage(vals, idxs, scrs)


# speedup vs baseline: 16.8827x; 1.0012x over previous
import jax
import jax.numpy as jnp
from jax import lax
from jax.experimental import pallas as pl
from jax.experimental.pallas import tpu as pltpu
from jax.experimental.pallas import tpu_sc as plsc

TOPK = 128
NCAND = 256
INV_LOG10 = 0.43429448190325176

N = 1_000_000
PADN = 1_048_576
CHUNK = PADN // 32
NV = CHUNK // 16
SEG_ELEMS = 256
NSEG = CHUNK // SEG_ELEMS
GRP_SEGS = 16
NGRP = NSEG // GRP_SEGS
NEG = jnp.float32(-jnp.inf)
BIGPOS = jnp.int32(1 << 30)


def _lane():
    return lax.iota(jnp.int32, 16)


def _take(v, idx):
    return v.at[idx].get(mode="promise_in_bounds")


def _argmax_tree(v, p):
    lane = _lane()
    cv, cp = v, p
    for k in (8, 4, 2, 1):
        sh = (lane + k) & 15
        pv = _take(cv, sh)
        pp = _take(cp, sh)
        better = jnp.logical_or(pv > cv, jnp.logical_and(pv == cv, pp < cp))
        cv = jnp.where(better, pv, cv)
        cp = jnp.where(better, pp, cp)
    vs = cv[0]
    ps = cp[0]
    match = jnp.logical_and(v == vs, p == ps)
    lid = jnp.where(match, lane, 16)
    for k in (8, 4, 2, 1):
        lid = jnp.minimum(lid, _take(lid, (lane + k) & 15))
    return vs, ps, lid[0]


def _argmax_tree3(v, p, i):
    lane = _lane()
    cv, cp, ci = v, p, i
    for k in (8, 4, 2, 1):
        sh = (lane + k) & 15
        pv = _take(cv, sh)
        pp = _take(cp, sh)
        pi = _take(ci, sh)
        better = jnp.logical_or(pv > cv, jnp.logical_and(pv == cv, pp < cp))
        cv = jnp.where(better, pv, cv)
        cp = jnp.where(better, pp, cp)
        ci = jnp.where(better, pi, ci)
    return cv[0], cp[0], ci[0]


def _scan_pairs(val_ref, pos_ref, base, count):
    def body(j, carry):
        mv, mp = carry
        v = val_ref[pl.ds((base + j) * 16, 16)]
        p = pos_ref[pl.ds((base + j) * 16, 16)]
        upd = jnp.logical_or(v > mv, jnp.logical_and(v == mv, p < mp))
        return jnp.where(upd, v, mv), jnp.where(upd, p, mp)
    return lax.fori_loop(0, count, body, (jnp.full((16,), NEG),
                                          jnp.full((16,), BIGPOS)))


def _sc_body(pnl_hbm, scores_hbm, ov_hbm, oi_hbm, os_hbm,
             data, mv1, mp1, gv2, gp2, tvv, tvi,
             cmv, cmi, cmsi, outv, outi, cidx, scrv, svv, svi, sem):
    c = lax.axis_index("c")
    s = lax.axis_index("s")
    w = c * 16 + s
    tilebase = w * CHUNK
    lane = _lane()

    pltpu.sync_copy(pnl_hbm.at[pl.ds(w * CHUNK, CHUNK)], data)

    def seg_build(seg, _):
        def inner(j, carry):
            mv, mp = carry
            pos0 = seg * SEG_ELEMS + j * 16
            v = data[pl.ds(pos0, 16)]
            p = pos0 + lane
            upd = jnp.logical_or(v > mv, jnp.logical_and(v == mv, p < mp))
            return jnp.where(upd, v, mv), jnp.where(upd, p, mp)
        mv, mp = lax.fori_loop(0, SEG_ELEMS // 16, inner,
                               (jnp.full((16,), NEG), jnp.full((16,), BIGPOS)))
        mv1[pl.ds(seg * 16, 16)] = mv
        mp1[pl.ds(seg * 16, 16)] = mp
        return 0
    lax.fori_loop(0, NSEG, seg_build, 0)

    def grp_build(g, _):
        gv, gp = _scan_pairs(mv1, mp1, g * GRP_SEGS, GRP_SEGS)
        gv2[pl.ds(g * 16, 16)] = gv
        gp2[pl.ds(g * 16, 16)] = gp
        return 0
    lax.fori_loop(0, NGRP, grp_build, 0)

    def extract(t, carry):
        wv, wi = carry
        m3v, m3p = _scan_pairs(gv2, gp2, 0, NGRP)
        vs, ps, _ = _argmax_tree(m3v, m3p)
        vo = (ps >> 4) << 4
        lv = ps & 15
        dv = data[pl.ds(vo, 16)]
        data[pl.ds(vo, 16)] = jnp.where(lane == lv, NEG, dv)
        sstar = ps >> 8
        gstar = ps >> 12
        def seg_rescan(j, carry):
            mv, mp = carry
            pos0 = sstar * SEG_ELEMS + j * 16
            v = data[pl.ds(pos0, 16)]
            p = pos0 + lane
            upd = jnp.logical_or(v > mv, jnp.logical_and(v == mv, p < mp))
            return jnp.where(upd, v, mv), jnp.where(upd, p, mp)
        mv, mp = lax.fori_loop(0, SEG_ELEMS // 16, seg_rescan,
                               (jnp.full((16,), NEG), jnp.full((16,), BIGPOS)))
        mv1[pl.ds(sstar * 16, 16)] = mv
        mp1[pl.ds(sstar * 16, 16)] = mp
        gv, gp = _scan_pairs(mv1, mp1, gstar * GRP_SEGS, GRP_SEGS)
        gv2[pl.ds(gstar * 16, 16)] = gv
        gp2[pl.ds(gstar * 16, 16)] = gp
        slot = t & 15
        wv = jnp.where(slot == 0, jnp.full((16,), NEG), wv)
        wi = jnp.where(slot == 0, jnp.zeros((16,), jnp.int32), wi)
        wv = jnp.where(lane == slot, vs, wv)
        wi = jnp.where(lane == slot, tilebase + ps, wi)
        tvv[pl.ds((t >> 4) << 4, 16)] = wv
        tvi[pl.ds((t >> 4) << 4, 16)] = wi
        return wv, wi
    lax.fori_loop(0, TOPK, extract,
                  (jnp.full((16,), NEG), jnp.zeros((16,), jnp.int32)))

    pltpu.sync_copy(tvv, svv.at[pl.ds(s * TOPK, TOPK)])
    pltpu.sync_copy(tvi, svi.at[pl.ds(s * TOPK, TOPK)])
    plsc.subcore_barrier()

    pltpu.sync_copy(svv, cmv)
    pltpu.sync_copy(svi, cmi)

    def mseg_build(seg, _):
        def inner(j, carry):
            mv, mp, mi = carry
            pos0 = seg * 256 + j * 16
            v = cmv[pl.ds(pos0, 16)]
            i = cmi[pl.ds(pos0, 16)]
            p = pos0 + lane
            upd = jnp.logical_or(v > mv, jnp.logical_and(v == mv, p < mp))
            return (jnp.where(upd, v, mv), jnp.where(upd, p, mp),
                    jnp.where(upd, i, mi))
        mv, mp, mi = lax.fori_loop(0, 16, inner,
                                   (jnp.full((16,), NEG),
                                    jnp.full((16,), BIGPOS),
                                    jnp.zeros((16,), jnp.int32)))
        gv2[pl.ds(seg * 16, 16)] = mv
        gp2[pl.ds(seg * 16, 16)] = mp
        cmsi[pl.ds(seg * 16, 16)] = mi
        return 0
    lax.fori_loop(0, 8, mseg_build, 0)

    def merge(t, carry):
        wv, wi = carry
        def topscan(j, carry):
            mv, mp, mi = carry
            v = gv2[pl.ds(j * 16, 16)]
            p = gp2[pl.ds(j * 16, 16)]
            i = cmsi[pl.ds(j * 16, 16)]
            upd = jnp.logical_or(v > mv, jnp.logical_and(v == mv, p < mp))
            return (jnp.where(upd, v, mv), jnp.where(upd, p, mp),
                    jnp.where(upd, i, mi))
        m3v, m3p, m3i = lax.fori_loop(0, 8, topscan,
                                      (jnp.full((16,), NEG),
                                       jnp.full((16,), BIGPOS),
                                       jnp.zeros((16,), jnp.int32)))
        vs, ps, is_ = _argmax_tree3(m3v, m3p, m3i)
        vo = (ps >> 4) << 4
        lv = ps & 15
        dv = cmv[pl.ds(vo, 16)]
        cmv[pl.ds(vo, 16)] = jnp.where(lane == lv, NEG, dv)
        sstar = ps >> 8
        def mseg_rescan(j, carry):
            mv, mp, mi = carry
            pos0 = sstar * 256 + j * 16
            v = cmv[pl.ds(pos0, 16)]
            i = cmi[pl.ds(pos0, 16)]
            p = pos0 + lane
            upd = jnp.logical_or(v > mv, jnp.logical_and(v == mv, p < mp))
            return (jnp.where(upd, v, mv), jnp.where(upd, p, mp),
                    jnp.where(upd, i, mi))
        mv, mp, mi = lax.fori_loop(0, 16, mseg_rescan,
                                   (jnp.full((16,), NEG),
                                    jnp.full((16,), BIGPOS),
                                    jnp.zeros((16,), jnp.int32)))
        gv2[pl.ds(sstar * 16, 16)] = mv
        gp2[pl.ds(sstar * 16, 16)] = mp
        cmsi[pl.ds(sstar * 16, 16)] = mi
        slot = t & 15
        wv = jnp.where(slot == 0, jnp.full((16,), NEG), wv)
        wi = jnp.where(slot == 0, jnp.zeros((16,), jnp.int32), wi)
        wv = jnp.where(lane == slot, vs, wv)
        wi = jnp.where(lane == slot, is_, wi)
        outv[pl.ds((t >> 4) << 4, 16)] = wv
        outi[pl.ds((t >> 4) << 4, 16)] = wi
        return wv, wi
    lax.fori_loop(0, TOPK, merge,
                  (jnp.full((16,), NEG), jnp.zeros((16,), jnp.int32)))

    def clampi(j, _):
        cidx[pl.ds(j * 16, 16)] = jnp.minimum(outi[pl.ds(j * 16, 16)], N - 1)
        return 0
    lax.fori_loop(0, TOPK // 16, clampi, 0)

    @pl.when(s == 0)
    def _():
        pltpu.async_copy(scores_hbm.at[cidx], scrv, sem).wait()
        pltpu.sync_copy(outv, ov_hbm.at[pl.ds(c * TOPK, TOPK)])
        pltpu.sync_copy(outi, oi_hbm.at[pl.ds(c * TOPK, TOPK)])
        pltpu.sync_copy(scrv, os_hbm.at[pl.ds(c * TOPK, TOPK)])


@jax.jit
def _candidate_stage(pnl_padded, scores):
    mesh = plsc.VectorSubcoreMesh(core_axis_name="c", subcore_axis_name="s")
    f = pl.kernel(
        _sc_body,
        out_type=(
            jax.ShapeDtypeStruct((NCAND,), jnp.float32),
            jax.ShapeDtypeStruct((NCAND,), jnp.int32),
            jax.ShapeDtypeStruct((NCAND,), jnp.float32),
        ),
        mesh=mesh,
        scratch_types=[
            pltpu.VMEM((CHUNK,), jnp.float32),
            pltpu.VMEM((NSEG * 16,), jnp.float32),
            pltpu.VMEM((NSEG * 16,), jnp.int32),
            pltpu.VMEM((NGRP * 16,), jnp.float32),
            pltpu.VMEM((NGRP * 16,), jnp.int32),
            pltpu.VMEM((TOPK,), jnp.float32),
            pltpu.VMEM((TOPK,), jnp.int32),
            pltpu.VMEM((16 * TOPK,), jnp.float32),
            pltpu.VMEM((16 * TOPK,), jnp.int32),
            pltpu.VMEM((NGRP * 16,), jnp.int32),
            pltpu.VMEM((TOPK,), jnp.float32),
            pltpu.VMEM((TOPK,), jnp.int32),
            pltpu.VMEM((TOPK,), jnp.int32),
            pltpu.VMEM((TOPK,), jnp.float32),
            pltpu.VMEM_SHARED((16 * TOPK,), jnp.float32),
            pltpu.VMEM_SHARED((16 * TOPK,), jnp.int32),
            pltpu.SemaphoreType.DMA,
        ],
    )
    return f(pnl_padded, scores)


def _loss_body(vr_ref, ir_ref, sr_ref, vc_ref, ic_ref, sc_ref, out_ref):
    vr = vr_ref[...]
    ir = ir_ref[...]
    sr = sr_ref[...]
    vc = vc_ref[...]
    ic = ic_ref[...]
    sc = sc_ref[...]

    beats_cr = jnp.logical_or(vr > vc, jnp.logical_and(vr == vc, ir < ic))
    r_col = jnp.sum(beats_cr.astype(jnp.float32), axis=1, keepdims=True)
    beats_rc = jnp.logical_or(vc > vr, jnp.logical_and(vc == vr, ic < ir))
    r_row = jnp.sum(beats_rc.astype(jnp.float32), axis=0, keepdims=True)

    kr = lax.broadcasted_iota(jnp.int32, (NCAND, TOPK), 1).astype(jnp.float32)
    P = (r_col == kr).astype(jnp.float32)
    kc = lax.broadcasted_iota(jnp.int32, (TOPK, NCAND), 0).astype(jnp.float32)
    Q = (kc == r_row).astype(jnp.float32)

    sv_row = jnp.dot(vr, P, preferred_element_type=jnp.float32)
    ss_row = jnp.dot(sr, P, preferred_element_type=jnp.float32)
    sv_col = jnp.dot(Q, vc, preferred_element_type=jnp.float32)
    ss_col = jnp.dot(Q, sc, preferred_element_type=jnp.float32)

    i_r = lax.broadcasted_iota(jnp.int32, (TOPK, TOPK), 1)
    i_c = lax.broadcasted_iota(jnp.int32, (TOPK, TOPK), 0)
    mask = (i_r > i_c).astype(jnp.float32)

    pnl_d = (sv_col - sv_row) * mask
    npd = jnp.log(pnl_d + 1.0) * INV_LOG10
    x = (ss_col - ss_row) * mask
    bce = jnp.maximum(x, 0.0) - x * mask + jnp.log(1.0 + jnp.exp(-jnp.abs(x)))
    total = jnp.sum(npd * bce) * (1.0 / TOPK)
    out_ref[...] = jnp.broadcast_to(total, (1, 1))


@jax.jit
def _loss_stage(vals, idxs, scrs):
    vr = vals.reshape(1, NCAND)
    ir = idxs.reshape(1, NCAND).astype(jnp.int32)
    sr = scrs.reshape(1, NCAND)
    vc = vals.reshape(NCAND, 1)
    ic = idxs.reshape(NCAND, 1).astype(jnp.int32)
    sc = scrs.reshape(NCAND, 1)
    out = pl.pallas_call(
        _loss_body,
        out_shape=jax.ShapeDtypeStruct((1, 1), jnp.float32),
    )(vr, ir, sr, vc, ic, sc)
    return out.reshape(())


def kernel(scores, target, pnl):
    del target
    pnl_padded = jnp.concatenate(
        [pnl, jnp.full((PADN - N,), -jnp.inf, jnp.float32)])
    vals, idxs, scrs = _candidate_stage(pnl_padded, scores)
    return _loss_stage(vals, idxs, scrs)
